# Initial kernel scaffold; baseline (speedup 1.0000x reference)
import jax
import jax.numpy as jnp
from jax.experimental import pallas as pl


def kernel(x):
    raise NotImplementedError("write your pallas kernel here")


# speedup vs baseline: 5.6041x; 5.6041x over previous
import jax
import jax.numpy as jnp
from jax import lax
from jax.experimental import pallas as pl

_BR = 256
_NCHUNK = 8
_CW = 128


def _scan_body(x_ref, o_ref):
    x = x_ref[...]
    row = lax.broadcasted_iota(jnp.int32, (_CW, _CW), 0)
    col = lax.broadcasted_iota(jnp.int32, (_CW, _CW), 1)
    u = (row < col).astype(jnp.float32)
    carry = jnp.zeros((_BR, 1), dtype=jnp.float32)
    for k in range(_NCHUNK):
        xc = x[:, k * _CW:(k + 1) * _CW]
        excl = jnp.dot(xc, u, preferred_element_type=jnp.float32)
        o_ref[:, k * _CW:(k + 1) * _CW] = excl + carry
        carry = carry + jnp.sum(xc, axis=1, keepdims=True)


def kernel(x):
    n_rows, n_cols = x.shape
    grid = (n_rows // _BR,)
    return pl.pallas_call(
        _scan_body,
        grid=grid,
        in_specs=[pl.BlockSpec((_BR, n_cols), lambda i: (i, 0))],
        out_specs=pl.BlockSpec((_BR, n_cols), lambda i: (i, 0)),
        out_shape=jax.ShapeDtypeStruct((n_rows, n_cols), jnp.float32),
    )(x)
